# Initial kernel scaffold; baseline (speedup 1.0000x reference)
import jax
import jax.numpy as jnp
from jax.experimental import pallas as pl


def kernel(x, target):
    raise NotImplementedError("write your pallas kernel here")


# speedup vs baseline: 1.7465x; 1.7465x over previous
import math

import jax
import jax.numpy as jnp
from jax import lax
from jax.experimental import pallas as pl
from jax.experimental.pallas import tpu as pltpu

_BATCH = 1024
_SIZE = 100000
_SMOOTHING = 0.1
_CONFIDENCE = 1.0 - _SMOOTHING
_FILL = _SMOOTHING / (_SIZE - 2)
_DELTA = _CONFIDENCE - _FILL
_CONST = _BATCH * ((_SIZE - 1) * _FILL * math.log(_FILL)
                   + _CONFIDENCE * math.log(_CONFIDENCE))

_W = 2048
_GRID = (_SIZE + _W - 1) // _W


def _loss_body(tgt_ref, x_ref, out_ref, acc_ref):
    j = pl.program_id(0)

    @pl.when(j == 0)
    def _init():
        acc_ref[0] = 0.0
        acc_ref[1] = 0.0

    xb = x_ref[...]
    ids = lax.broadcasted_iota(jnp.int32, (_BATCH, _W), 1) + j * _W
    xv = jnp.where(ids < _SIZE, xb, 0.0)
    acc_ref[0] += jnp.sum(xv)
    hit = ids == tgt_ref[...]
    acc_ref[1] += jnp.sum(jnp.where(hit, xv, 0.0))

    @pl.when(j == _GRID - 1)
    def _fin():
        loss = _CONST - _FILL * acc_ref[0] - _DELTA * acc_ref[1]
        out_ref[0, 0] = loss.astype(jnp.float32)


@jax.jit
def kernel(x, target):
    tgt = target.reshape(_BATCH, 1)
    out = pl.pallas_call(
        _loss_body,
        grid=(_GRID,),
        in_specs=[
            pl.BlockSpec((_BATCH, 1), lambda j: (0, 0)),
            pl.BlockSpec((_BATCH, _W), lambda j: (0, j)),
        ],
        out_specs=pl.BlockSpec(memory_space=pltpu.SMEM),
        out_shape=jax.ShapeDtypeStruct((1, 1), jnp.float32),
        scratch_shapes=[pltpu.SMEM((2,), jnp.float32)],
    )(tgt, x)
    return out[0, 0]
